# baseline (speedup 1.0000x reference)
import jax
import jax.numpy as jnp
from jax import lax
from jax.experimental import pallas as pl
from jax.experimental.pallas import tpu as pltpu
from jax.experimental.pallas import tpu_sc as plsc

VOCAB_SIZE = 100000
EMBEDDING_SIZE = 64
BATCH = 4096
SEQ_LEN = 200

NUM_WORKERS = 32
BATCH_PER_W = BATCH // NUM_WORKERS
NSLOT = 5
S_STRIDE = 8 * NUM_WORKERS * 8 * 128
EG_STRIDE = NUM_WORKERS * 8 * 128


def _embed_kernel(idx_hbm, table_hbm, pe_hbm, out_hbm,
                  slab, idx_t, pe_v,
                  rows0, rows1, rows2, rows3, rows4,
                  g0, g1, g2, g3, g4, w0, w1, w2, w3, w4):
    nc = 2
    wid = lax.axis_index("s") * nc + lax.axis_index("c")
    base = wid * BATCH_PER_W

    rows = (rows0, rows1, rows2, rows3, rows4)
    gsem = (g0, g1, g2, g3, g4)
    wsem = (w0, w1, w2, w3, w4)

    pltpu.sync_copy(idx_hbm.at[pl.ds(base * SEQ_LEN, BATCH_PER_W * SEQ_LEN)],
                    slab.at[pl.ds(0, BATCH_PER_W * SEQ_LEN)])
    pltpu.sync_copy(pe_hbm, pe_v)

    lanes = lax.iota(jnp.int32, 16)

    sv128 = [(lanes + 16 * k) * BATCH_PER_W for k in range(13)]
    tail = lanes < 8

    def t_body(p, _):
        off = p * SEQ_LEN
        for k in range(12):
            v = slab[pl.ds(off + 16 * k, 16)]
            plsc.store_scatter(idx_t, [sv128[k] + p], v)
        v = slab[pl.ds(off + 192, 16)]
        plsc.store_scatter(idx_t, [sv128[12] + p], v, mask=tail)
        return ()

    lax.fori_loop(0, BATCH_PER_W, t_body, (), unroll=2)

    def g_start(s, b):
        pltpu.async_copy(
            table_hbm.at[idx_t.at[pl.ds(s * BATCH_PER_W, BATCH_PER_W)]],
            rows[b], gsem[b])

    def g_wait(b):
        pltpu.make_async_copy(
            table_hbm.at[idx_t.at[pl.ds(0, BATCH_PER_W)]], rows[b],
            gsem[b]).wait()

    def w_start(s, b):
        woff = s * S_STRIDE + wid * (8 * 128)

        def e_body(e, _):
            off = woff + (e // 8) * EG_STRIDE + (e % 8) * 128
            pltpu.async_copy(rows[b].at[:, pl.ds(e, 1)],
                             out_hbm.at[pl.ds(off, BATCH_PER_W), :],
                             wsem[b])
            return ()

        lax.fori_loop(0, EMBEDDING_SIZE, e_body, (), unroll=8)

    def w_drain(b):
        def e_body(e, _):
            pltpu.make_async_copy(rows[b].at[:, pl.ds(0, 1)],
                                  out_hbm.at[pl.ds(0, BATCH_PER_W), :],
                                  wsem[b]).wait()
            return ()

        lax.fori_loop(0, EMBEDDING_SIZE, e_body, (), unroll=8)

    def pe_add(s, b):
        rows_r = rows[b]
        pe4 = [pe_v[s, pl.ds(16 * k, 16)] for k in range(4)]

        def t_loop(t, _):
            for k in range(4):
                plsc.addupdate(rows_r.at[t, pl.ds(16 * k, 16)], pe4[k])
            return ()

        lax.fori_loop(0, BATCH_PER_W, t_loop, (), unroll=4)

    for s in range(3):
        g_start(s, s)

    def outer(p, _):
        for q in range(NSLOT):
            s = NSLOT * p + q
            g_wait(q)

            @pl.when(s >= 2)
            def _():
                w_drain((q + 3) % NSLOT)

            @pl.when(s + 3 < SEQ_LEN)
            def _():
                g_start(s + 3, (q + 3) % NSLOT)

            pe_add(s, q)
            w_start(s, q)
        return ()

    lax.fori_loop(0, SEQ_LEN // NSLOT, outer, ())

    w_drain(3)
    w_drain(4)


@jax.jit
def _run(idx_flat, table, pe):
    mesh = plsc.VectorSubcoreMesh(core_axis_name="c", subcore_axis_name="s")
    fn = pl.kernel(
        _embed_kernel,
        mesh=mesh,
        compiler_params=pltpu.CompilerParams(use_tc_tiling_on_sc=False,
                                             needs_layout_passes=False),
        out_type=jax.ShapeDtypeStruct(
            (BATCH * SEQ_LEN * EMBEDDING_SIZE, 1), jnp.float32),
        scratch_types=[
            pltpu.VMEM((BATCH_PER_W * SEQ_LEN + 16,), jnp.int32),
            pltpu.VMEM((SEQ_LEN * BATCH_PER_W,), jnp.int32),
            pltpu.VMEM((SEQ_LEN, EMBEDDING_SIZE), jnp.float32),
        ] + [
            pltpu.VMEM((BATCH_PER_W, EMBEDDING_SIZE), jnp.float32)
            for _ in range(NSLOT)
        ] + [pltpu.SemaphoreType.DMA for _ in range(2 * NSLOT)],
    )
    raw = fn(idx_flat, table, pe)
    out5 = raw.reshape(SEQ_LEN, 8, NUM_WORKERS, 8, 128)
    return out5.transpose(2, 4, 0, 1, 3).reshape(BATCH, SEQ_LEN,
                                                 EMBEDDING_SIZE)


def kernel(inputs, word_embedding, position_embedding):
    idx_flat = inputs.astype(jnp.int32).reshape(BATCH * SEQ_LEN)
    pe = position_embedding[:SEQ_LEN]
    return _run(idx_flat, word_embedding, pe)


# speedup vs baseline: 169.1635x; 169.1635x over previous
import jax
import jax.numpy as jnp
from jax import lax
from jax.experimental import pallas as pl
from jax.experimental.pallas import tpu as pltpu
from jax.experimental.pallas import tpu_sc as plsc

VOCAB_SIZE = 100000
EMBEDDING_SIZE = 64
BATCH = 4096
SEQ_LEN = 200

NUM_WORKERS = 32
SEQS_PER_WORKER = BATCH // NUM_WORKERS
NBUF = 4
HALF = EMBEDDING_SIZE // 2


def _embed_kernel(idx_hbm, table_hbm, pe_hbm, out_hbm,
                  pe_v,
                  ix0, ix1, ix2, ix3,
                  rows0, rows1, rows2, rows3,
                  i0, i1, i2, i3, g0, g1, g2, g3, w0, w1, w2, w3):
    nc = 2
    wid = lax.axis_index("s") * nc + lax.axis_index("c")
    base_seq = wid * SEQS_PER_WORKER

    pltpu.sync_copy(pe_hbm, pe_v)

    idx_bufs = (ix0, ix1, ix2, ix3)
    row_bufs = (rows0, rows1, rows2, rows3)
    i_sems = (i0, i1, i2, i3)
    g_sems = (g0, g1, g2, g3)
    w_sems = (w0, w1, w2, w3)

    def idx_fetch(c, b):
        s = base_seq + c
        pltpu.async_copy(idx_hbm.at[pl.ds(s * SEQ_LEN, SEQ_LEN)],
                         idx_bufs[b], i_sems[b])

    def idx_wait(b):
        pltpu.make_async_copy(idx_hbm.at[pl.ds(0, SEQ_LEN)],
                              idx_bufs[b], i_sems[b]).wait()

    def fetch(b):
        pltpu.async_copy(table_hbm.at[idx_bufs[b]], row_bufs[b], g_sems[b])

    def gather_wait(b):
        pltpu.make_async_copy(table_hbm.at[idx_bufs[b]], row_bufs[b],
                              g_sems[b]).wait()

    def wb_wait(b):
        pltpu.make_async_copy(row_bufs[b], out_hbm.at[base_seq],
                              w_sems[b]).wait()

    def add_pe(b):
        rows = row_bufs[b]

        def body(r, _):
            for h in range(8):
                a = h // 4
                sl = pl.ds((h % 4) * 16, 16)
                plsc.addupdate(rows.at[2 * r + a, sl],
                               pe_v[r, pl.ds(h * 16, 16)])
            return ()

        lax.fori_loop(0, SEQ_LEN // 2, body, (), unroll=4)

    for b in range(NBUF):
        idx_fetch(b, b)
    for b in range(2):
        idx_wait(b)
        fetch(b)

    def outer(p, _):
        for b in range(NBUF):
            c = p * NBUF + b
            gather_wait(b)

            @pl.when(c + 4 < SEQS_PER_WORKER)
            def _():
                idx_fetch(c + 4, b)

            tb = (b + 2) % NBUF

            @pl.when(c >= 2)
            def _():
                wb_wait(tb)

            @pl.when(c + 2 < SEQS_PER_WORKER)
            def _():
                idx_wait(tb)
                fetch(tb)

            add_pe(b)
            pltpu.async_copy(row_bufs[b], out_hbm.at[base_seq + c],
                             w_sems[b])
        return ()

    lax.fori_loop(0, SEQS_PER_WORKER // NBUF, outer, ())

    wb_wait(2)
    wb_wait(3)


@jax.jit
def _run(idx_flat, table, pe_packed):
    mesh = plsc.VectorSubcoreMesh(core_axis_name="c", subcore_axis_name="s")
    fn = pl.kernel(
        _embed_kernel,
        mesh=mesh,
        compiler_params=pltpu.CompilerParams(use_tc_tiling_on_sc=False),
        out_type=jax.ShapeDtypeStruct((BATCH, SEQ_LEN, EMBEDDING_SIZE),
                                      jnp.float32),
        scratch_types=[
            pltpu.VMEM((SEQ_LEN // 2, 2 * EMBEDDING_SIZE), jnp.float32),
        ] + [
            pltpu.VMEM((SEQ_LEN,), jnp.int32) for _ in range(NBUF)
        ] + [
            pltpu.VMEM((SEQ_LEN, EMBEDDING_SIZE), jnp.float32)
            for _ in range(NBUF)
        ] + [pltpu.SemaphoreType.DMA for _ in range(3 * NBUF)],
    )
    return fn(idx_flat, table, pe_packed)


def kernel(inputs, word_embedding, position_embedding):
    idx_flat = inputs.astype(jnp.int32).reshape(BATCH * SEQ_LEN)
    pe_packed = position_embedding[:SEQ_LEN].reshape(SEQ_LEN // 2,
                                                     2 * EMBEDDING_SIZE)
    return _run(idx_flat, word_embedding, pe_packed)
